# Initial kernel scaffold; baseline (speedup 1.0000x reference)
import jax
import jax.numpy as jnp
from jax.experimental import pallas as pl


def kernel(ui_edge_index, ui_edge_weight, uu_edge_index, uu_edge_weight, user_feat, item_feat, Wu1, bu1, Wu2, bu2, Wi1, bi1, Wi2, bi2, Wm, bm):
    raise NotImplementedError("write your pallas kernel here")


# speedup vs baseline: 3.4634x; 3.4634x over previous
import functools

import jax
import jax.numpy as jnp
from jax import lax
from jax.experimental import pallas as pl
from jax.experimental.pallas import tpu as pltpu
from jax.experimental.pallas import tpu_sc as plsc

N_USERS = 2000
N_ITEMS = 8000
N_ALL = N_USERS + N_ITEMS
HID = 128

NC = 2
NS = 16
NW = NC * NS
CHUNK = 128


def _mlp_body(x_ref, w1_ref, b1_ref, w2_ref, b2_ref, o_ref):
    h = jnp.maximum(
        jnp.dot(x_ref[...], w1_ref[0], preferred_element_type=jnp.float32)
        + b1_ref[0], 0.0)
    o_ref[...] = (
        jnp.dot(h, w2_ref[0], preferred_element_type=jnp.float32) + b2_ref[0])


def _mlp_call(x, W1, b1, W2, b2):
    wmap = lambda i: (jnp.minimum(i, jnp.int32(1)), jnp.int32(0), jnp.int32(0))
    return pl.pallas_call(
        _mlp_body,
        grid=(5,),
        in_specs=[
            pl.BlockSpec((2000, HID), lambda i: (i, jnp.int32(0))),
            pl.BlockSpec((1, HID, HID), wmap),
            pl.BlockSpec((1, 1, HID), wmap),
            pl.BlockSpec((1, HID, HID), wmap),
            pl.BlockSpec((1, 1, HID), wmap),
        ],
        out_specs=pl.BlockSpec((2000, HID), lambda i: (i, jnp.int32(0))),
        out_shape=jax.ShapeDtypeStruct((N_ALL, HID), jnp.float32),
    )(x, W1, b1.reshape(2, 1, HID), W2, b2.reshape(2, 1, HID))


def _make_spmm(n_nodes, e_pad):
    epw = e_pad // NW
    n_chunks = epw // CHUNK
    rps = n_nodes // NS
    n_full = rps // CHUNK
    rem = rps % CHUNK
    mesh = plsc.VectorSubcoreMesh(core_axis_name="c", subcore_axis_name="s")

    @functools.partial(
        pl.kernel,
        mesh=mesh,
        out_type=jax.ShapeDtypeStruct((NC, n_nodes, HID), jnp.float32),
        scratch_types=[
            pltpu.VMEM((CHUNK,), jnp.int32),
            pltpu.VMEM((CHUNK,), jnp.int32),
            pltpu.VMEM((CHUNK,), jnp.float32),
            pltpu.VMEM((CHUNK, HID), jnp.float32),
            pltpu.VMEM_SHARED((n_nodes, HID), jnp.float32),
            pltpu.SemaphoreType.DMA,
        ],
    )
    def spmm(src_hbm, dst_hbm, w_hbm, h_hbm, out_hbm,
             src_v, dst_v, w_v, rows_v, acc, sem):
        i32 = jnp.int32
        c = lax.axis_index("c").astype(i32)
        s = lax.axis_index("s").astype(i32)
        gid = c * i32(NS) + s

        def zero_row(i, carry):
            for j in range(HID // 16):
                rows_v[i, pl.ds(j * 16, 16)] = jnp.zeros((16,), jnp.float32)
            return carry
        lax.fori_loop(i32(0), i32(CHUNK), zero_row, i32(0))
        base_row = s * i32(rps)
        for k in range(n_full):
            pltpu.sync_copy(rows_v, acc.at[pl.ds(base_row + k * CHUNK, CHUNK)])
        if rem:
            pltpu.sync_copy(rows_v.at[pl.ds(0, rem)],
                            acc.at[pl.ds(base_row + n_full * CHUNK, rem)])
        plsc.subcore_barrier()

        ebase = gid * i32(epw)

        def do_chunk(i, carry):
            off = ebase + i * i32(CHUNK)
            pltpu.sync_copy(src_hbm.at[pl.ds(off, CHUNK)], src_v)
            pltpu.sync_copy(dst_hbm.at[pl.ds(off, CHUNK)], dst_v)
            pltpu.sync_copy(w_hbm.at[pl.ds(off, CHUNK)], w_v)
            pltpu.async_copy(h_hbm.at[src_v], rows_v, sem).wait()

            def scale_group(g, carry2):
                w16 = w_v[pl.ds(g * 16, 16)]
                for l in range(16):
                    wv = jnp.full((16,), w16[l], jnp.float32)
                    e = g * 16 + l
                    for j in range(HID // 16):
                        sl = pl.ds(j * 16, 16)
                        rows_v[e, sl] = rows_v[e, sl] * wv
                return carry2
            lax.fori_loop(i32(0), i32(CHUNK // 16), scale_group, i32(0))
            pltpu.sync_copy(rows_v, acc.at[dst_v], add=True)
            return carry
        lax.fori_loop(i32(0), i32(n_chunks), do_chunk, i32(0))
        plsc.subcore_barrier()

        for k in range(n_full):
            sl = pl.ds(base_row + k * CHUNK, CHUNK)
            pltpu.sync_copy(acc.at[sl], rows_v)
            pltpu.sync_copy(rows_v, out_hbm.at[c, sl])
        if rem:
            sl = pl.ds(base_row + n_full * CHUNK, rem)
            pltpu.sync_copy(acc.at[sl], rows_v.at[pl.ds(0, rem)])
            pltpu.sync_copy(rows_v.at[pl.ds(0, rem)], out_hbm.at[c, sl])

    return spmm


_E_PAD_UI = 323584
_E_PAD_UU = 32768
N_ALL_PAD = 10240
N_USERS_PAD = 2048
_spmm_ui = _make_spmm(N_ALL_PAD, _E_PAD_UI)
_spmm_uu = _make_spmm(N_USERS_PAD, _E_PAD_UU)


def _pad_edges(edge_index, edge_weight, e_pad):
    e = edge_index.shape[1]
    pad = e_pad - e
    src = jnp.pad(edge_index[0].astype(jnp.int32), (0, pad))
    dst = jnp.pad(edge_index[1].astype(jnp.int32), (0, pad))
    w = jnp.pad(edge_weight.astype(jnp.float32), (0, pad))
    return src, dst, w


def _add2_body(p_ref, o_ref):
    o_ref[...] = p_ref[0] + p_ref[1]


def _add2(p):
    n = p.shape[1]
    return pl.pallas_call(
        _add2_body,
        grid=(n // 1024,),
        in_specs=[pl.BlockSpec((2, 1024, HID), lambda i: (jnp.int32(0), i, jnp.int32(0)))],
        out_specs=pl.BlockSpec((1024, HID), lambda i: (i, jnp.int32(0))),
        out_shape=jax.ShapeDtypeStruct((n, HID), jnp.float32),
    )(p)


def _mean3_body(a_ref, b_ref, p_ref, o_ref):
    o_ref[...] = (a_ref[...] + b_ref[...] + p_ref[0] + p_ref[1]) * (1.0 / 3.0)


def _mean3(a, b, p):
    n = a.shape[0]
    return pl.pallas_call(
        _mean3_body,
        grid=(n // 1024,),
        in_specs=[
            pl.BlockSpec((1024, HID), lambda i: (i, jnp.int32(0))),
            pl.BlockSpec((1024, HID), lambda i: (i, jnp.int32(0))),
            pl.BlockSpec((2, 1024, HID), lambda i: (jnp.int32(0), i, jnp.int32(0))),
        ],
        out_specs=pl.BlockSpec((1024, HID), lambda i: (i, jnp.int32(0))),
        out_shape=jax.ShapeDtypeStruct((n, HID), jnp.float32),
    )(a, b, p)


def _final_body(ego_ref, ue_ref, pu_ref, wm_ref, bm_ref, o_ref):
    uu = (ue_ref[...] + pu_ref[0] + pu_ref[1]) * 0.5
    o_ref[...] = (
        jnp.dot(ego_ref[...], wm_ref[0:HID], preferred_element_type=jnp.float32)
        + jnp.dot(ue_ref[...], wm_ref[HID:2 * HID],
                  preferred_element_type=jnp.float32)
        + jnp.dot(uu, wm_ref[2 * HID:3 * HID],
                  preferred_element_type=jnp.float32)
        + bm_ref[...])


def _final(ego, ue, pu, Wm, bm):
    return pl.pallas_call(
        _final_body,
        grid=(1,),
        in_specs=[
            pl.BlockSpec((N_USERS, HID), lambda i: (jnp.int32(0), jnp.int32(0))),
            pl.BlockSpec((N_USERS, HID), lambda i: (jnp.int32(0), jnp.int32(0))),
            pl.BlockSpec((2, N_USERS, HID), lambda i: (jnp.int32(0), jnp.int32(0), jnp.int32(0))),
            pl.BlockSpec((3 * HID, HID), lambda i: (jnp.int32(0), jnp.int32(0))),
            pl.BlockSpec((1, HID), lambda i: (jnp.int32(0), jnp.int32(0))),
        ],
        out_specs=pl.BlockSpec((N_USERS, HID), lambda i: (jnp.int32(0), jnp.int32(0))),
        out_shape=jax.ShapeDtypeStruct((N_USERS, HID), jnp.float32),
    )(ego, ue, pu, Wm, bm.reshape(1, HID))


def kernel(ui_edge_index, ui_edge_weight, uu_edge_index, uu_edge_weight,
           user_feat, item_feat, Wu1, bu1, Wu2, bu2, Wi1, bi1, Wi2, bi2,
           Wm, bm):
    x = jnp.concatenate([user_feat, item_feat], axis=0)
    W1 = jnp.stack([Wu1, Wi1])
    b1 = jnp.stack([bu1, bi1])
    W2 = jnp.stack([Wu2, Wi2])
    b2 = jnp.stack([bu2, bi2])
    all0 = _mlp_call(x, W1, b1, W2, b2)
    user_ego = all0[:N_USERS]
    item_ego = all0[N_USERS:]

    src, dst, w = _pad_edges(ui_edge_index, ui_edge_weight, _E_PAD_UI)
    all0p = jnp.pad(all0, ((0, N_ALL_PAD - N_ALL), (0, 0)))
    p1 = _spmm_ui(src, dst, w, all0p)
    h1 = _add2(p1)
    p2 = _spmm_ui(src, dst, w, h1)
    amep = _mean3(all0p, h1, p2)
    user_emb = amep[:N_USERS]
    item_emb = amep[N_USERS:N_ALL]

    usrc, udst, uw = _pad_edges(uu_edge_index, uu_edge_weight, _E_PAD_UU)
    uep = jnp.pad(user_emb, ((0, N_USERS_PAD - N_USERS), (0, 0)))
    pu = _spmm_uu(usrc, udst, uw, uep)
    user_final = _final(user_ego, user_emb, pu[:, :N_USERS], Wm, bm)
    return (user_final, item_emb, user_ego, item_ego)


# speedup vs baseline: 3.4793x; 1.0046x over previous
import functools

import jax
import jax.numpy as jnp
from jax import lax
from jax.experimental import pallas as pl
from jax.experimental.pallas import tpu as pltpu
from jax.experimental.pallas import tpu_sc as plsc

N_USERS = 2000
N_ITEMS = 8000
N_ALL = N_USERS + N_ITEMS
HID = 128

NC = 2
NS = 16
NW = NC * NS
CHUNK = 64


def _mlp_body(x_ref, w1_ref, b1_ref, w2_ref, b2_ref, o_ref):
    h = jnp.maximum(
        jnp.dot(x_ref[...], w1_ref[0], preferred_element_type=jnp.float32)
        + b1_ref[0], 0.0)
    o_ref[...] = (
        jnp.dot(h, w2_ref[0], preferred_element_type=jnp.float32) + b2_ref[0])


def _mlp_call(x, W1, b1, W2, b2):
    wmap = lambda i: (jnp.minimum(i, jnp.int32(1)), jnp.int32(0), jnp.int32(0))
    return pl.pallas_call(
        _mlp_body,
        grid=(5,),
        in_specs=[
            pl.BlockSpec((2000, HID), lambda i: (i, jnp.int32(0))),
            pl.BlockSpec((1, HID, HID), wmap),
            pl.BlockSpec((1, 1, HID), wmap),
            pl.BlockSpec((1, HID, HID), wmap),
            pl.BlockSpec((1, 1, HID), wmap),
        ],
        out_specs=pl.BlockSpec((2000, HID), lambda i: (i, jnp.int32(0))),
        out_shape=jax.ShapeDtypeStruct((N_ALL, HID), jnp.float32),
    )(x, W1, b1.reshape(2, 1, HID), W2, b2.reshape(2, 1, HID))


def _make_spmm(n_nodes, e_pad):
    epw = e_pad // NW
    n_chunks = epw // CHUNK
    n_pairs = n_chunks // 2
    rps = n_nodes // NS
    n_full = rps // CHUNK
    rem = rps % CHUNK
    mesh = plsc.VectorSubcoreMesh(core_axis_name="c", subcore_axis_name="s")

    @functools.partial(
        pl.kernel,
        mesh=mesh,
        out_type=jax.ShapeDtypeStruct((NC, n_nodes, HID), jnp.float32),
        scratch_types=[
            pltpu.VMEM((2, CHUNK), jnp.int32),
            pltpu.VMEM((2, CHUNK), jnp.int32),
            pltpu.VMEM((CHUNK, 16), jnp.float32),
            pltpu.VMEM((CHUNK, 16), jnp.float32),
            pltpu.VMEM((CHUNK, HID), jnp.float32),
            pltpu.VMEM((CHUNK, HID), jnp.float32),
            pltpu.VMEM_SHARED((n_nodes, HID), jnp.float32),
            pltpu.SemaphoreType.DMA,
            pltpu.SemaphoreType.DMA,
            pltpu.SemaphoreType.DMA,
            pltpu.SemaphoreType.DMA,
            pltpu.SemaphoreType.DMA,
            pltpu.SemaphoreType.DMA,
        ],
    )
    def spmm(meta_hbm, wexp_hbm, h_hbm, out_hbm,
             meta0, meta1, wexp0, wexp1, rows0, rows1, acc,
             msem0, msem1, gsem0, gsem1, ssem0, ssem1):
        i32 = jnp.int32
        c = lax.axis_index("c").astype(i32)
        s = lax.axis_index("s").astype(i32)
        gid = c * i32(NS) + s
        meta = (meta0, meta1)
        wexp = (wexp0, wexp1)
        rows = (rows0, rows1)
        msem = (msem0, msem1)
        gsem = (gsem0, gsem1)
        ssem = (ssem0, ssem1)

        def meta_start(b, cid):
            pltpu.async_copy(meta_hbm.at[cid], meta[b], msem[b])
            pltpu.async_copy(wexp_hbm.at[cid], wexp[b], msem[b])

        def meta_wait(b):
            pltpu.make_async_copy(meta_hbm.at[i32(0)], meta[b], msem[b]).wait()
            pltpu.make_async_copy(wexp_hbm.at[i32(0)], wexp[b], msem[b]).wait()

        def gather_start(b):
            pltpu.async_copy(h_hbm.at[meta[b].at[i32(0)]], rows[b], gsem[b])

        def gather_wait(b):
            pltpu.make_async_copy(h_hbm.at[meta[b].at[i32(0)]], rows[b],
                                  gsem[b]).wait()

        def scatter_start(b):
            pltpu.async_copy(rows[b], acc.at[meta[b].at[i32(1)]], ssem[b], add=True)

        def scatter_wait(b):
            pltpu.make_async_copy(rows[b], acc.at[meta[b].at[i32(1)]],
                                  ssem[b]).wait()

        def scale(b):
            rows_b = rows[b]
            wexp_b = wexp[b]

            @plsc.parallel_loop(jnp.int32(0), jnp.int32(CHUNK), jnp.int32(1), unroll=4)
            def _(e):
                wv = wexp_b[e, :]
                for j in range(HID // 16):
                    sl = pl.ds(j * 16, 16)
                    rows_b[e, sl] = rows_b[e, sl] * wv

        def zero_row(i, carry):
            for j in range(HID // 16):
                rows0[i, pl.ds(j * 16, 16)] = jnp.zeros((16,), jnp.float32)
            return carry
        lax.fori_loop(i32(0), i32(CHUNK), zero_row, i32(0))
        base_row = s * i32(rps)
        for k in range(n_full):
            pltpu.sync_copy(rows0, acc.at[pl.ds(base_row + k * CHUNK, CHUNK)])
        if rem:
            pltpu.sync_copy(rows0.at[pl.ds(0, rem)],
                            acc.at[pl.ds(base_row + n_full * CHUNK, rem)])
        plsc.subcore_barrier()

        cbase = gid * i32(n_chunks)

        meta_start(0, cbase)
        meta_start(1, cbase + i32(1))
        meta_wait(0)
        gather_start(0)

        def do_pair(g, carry):
            i0 = i32(2) * g
            gather_wait(0)
            meta_wait(1)
            gather_start(1)
            scale(0)
            scatter_start(0)
            gather_wait(1)
            scale(1)
            scatter_start(1)
            scatter_wait(0)
            nxt0 = jnp.where(i0 + i32(2) < i32(n_chunks), i0 + i32(2), i32(0))
            meta_start(0, cbase + nxt0)
            meta_wait(0)
            gather_start(0)
            scatter_wait(1)
            nxt1 = jnp.where(i0 + i32(3) < i32(n_chunks), i0 + i32(3), i32(1))
            meta_start(1, cbase + nxt1)
            return carry
        lax.fori_loop(i32(0), i32(n_pairs), do_pair, i32(0))
        gather_wait(0)
        meta_wait(1)
        plsc.subcore_barrier()

        for k in range(n_full):
            sl = pl.ds(base_row + k * CHUNK, CHUNK)
            pltpu.sync_copy(acc.at[sl], rows0)
            pltpu.sync_copy(rows0, out_hbm.at[c, sl])
        if rem:
            sl = pl.ds(base_row + n_full * CHUNK, rem)
            pltpu.sync_copy(acc.at[sl], rows0.at[pl.ds(0, rem)])
            pltpu.sync_copy(rows0.at[pl.ds(0, rem)], out_hbm.at[c, sl])

    return spmm


_E_PAD_UI = 323584
_E_PAD_UU = 32768
N_ALL_PAD = 10240
N_USERS_PAD = 2048
_spmm_ui = _make_spmm(N_ALL_PAD, _E_PAD_UI)
_spmm_uu = _make_spmm(N_USERS_PAD, _E_PAD_UU)


def _pad_edges(edge_index, edge_weight, e_pad):
    e = edge_index.shape[1]
    pad = e_pad - e
    nch = e_pad // CHUNK
    src = jnp.pad(edge_index[0].astype(jnp.int32), (0, pad)).reshape(nch, 1, CHUNK)
    dst = jnp.pad(edge_index[1].astype(jnp.int32), (0, pad)).reshape(nch, 1, CHUNK)
    meta = jnp.concatenate([src, dst], axis=1)
    w = jnp.pad(edge_weight.astype(jnp.float32), (0, pad))
    wexp = jnp.broadcast_to(w[:, None], (e_pad, 16)).reshape(nch, CHUNK, 16)
    return meta, wexp


def _add2_body(p_ref, o_ref):
    o_ref[...] = p_ref[0] + p_ref[1]


def _add2(p):
    n = p.shape[1]
    return pl.pallas_call(
        _add2_body,
        grid=(n // 1024,),
        in_specs=[pl.BlockSpec((2, 1024, HID), lambda i: (jnp.int32(0), i, jnp.int32(0)))],
        out_specs=pl.BlockSpec((1024, HID), lambda i: (i, jnp.int32(0))),
        out_shape=jax.ShapeDtypeStruct((n, HID), jnp.float32),
    )(p)


def _mean3_body(a_ref, b_ref, p_ref, o_ref):
    o_ref[...] = (a_ref[...] + b_ref[...] + p_ref[0] + p_ref[1]) * (1.0 / 3.0)


def _mean3(a, b, p):
    n = a.shape[0]
    return pl.pallas_call(
        _mean3_body,
        grid=(n // 1024,),
        in_specs=[
            pl.BlockSpec((1024, HID), lambda i: (i, jnp.int32(0))),
            pl.BlockSpec((1024, HID), lambda i: (i, jnp.int32(0))),
            pl.BlockSpec((2, 1024, HID), lambda i: (jnp.int32(0), i, jnp.int32(0))),
        ],
        out_specs=pl.BlockSpec((1024, HID), lambda i: (i, jnp.int32(0))),
        out_shape=jax.ShapeDtypeStruct((n, HID), jnp.float32),
    )(a, b, p)


def _final_body(ego_ref, ue_ref, pu_ref, wm_ref, bm_ref, o_ref):
    uu = (ue_ref[...] + pu_ref[0] + pu_ref[1]) * 0.5
    o_ref[...] = (
        jnp.dot(ego_ref[...], wm_ref[0:HID], preferred_element_type=jnp.float32)
        + jnp.dot(ue_ref[...], wm_ref[HID:2 * HID],
                  preferred_element_type=jnp.float32)
        + jnp.dot(uu, wm_ref[2 * HID:3 * HID],
                  preferred_element_type=jnp.float32)
        + bm_ref[...])


def _final(ego, ue, pu, Wm, bm):
    return pl.pallas_call(
        _final_body,
        grid=(1,),
        in_specs=[
            pl.BlockSpec((N_USERS, HID), lambda i: (jnp.int32(0), jnp.int32(0))),
            pl.BlockSpec((N_USERS, HID), lambda i: (jnp.int32(0), jnp.int32(0))),
            pl.BlockSpec((2, N_USERS, HID), lambda i: (jnp.int32(0), jnp.int32(0), jnp.int32(0))),
            pl.BlockSpec((3 * HID, HID), lambda i: (jnp.int32(0), jnp.int32(0))),
            pl.BlockSpec((1, HID), lambda i: (jnp.int32(0), jnp.int32(0))),
        ],
        out_specs=pl.BlockSpec((N_USERS, HID), lambda i: (jnp.int32(0), jnp.int32(0))),
        out_shape=jax.ShapeDtypeStruct((N_USERS, HID), jnp.float32),
    )(ego, ue, pu, Wm, bm.reshape(1, HID))


def kernel(ui_edge_index, ui_edge_weight, uu_edge_index, uu_edge_weight,
           user_feat, item_feat, Wu1, bu1, Wu2, bu2, Wi1, bi1, Wi2, bi2,
           Wm, bm):
    x = jnp.concatenate([user_feat, item_feat], axis=0)
    W1 = jnp.stack([Wu1, Wi1])
    b1 = jnp.stack([bu1, bi1])
    W2 = jnp.stack([Wu2, Wi2])
    b2 = jnp.stack([bu2, bi2])
    all0 = _mlp_call(x, W1, b1, W2, b2)
    user_ego = all0[:N_USERS]
    item_ego = all0[N_USERS:]

    meta, wexp = _pad_edges(ui_edge_index, ui_edge_weight, _E_PAD_UI)
    all0p = jnp.pad(all0, ((0, N_ALL_PAD - N_ALL), (0, 0)))
    p1 = _spmm_ui(meta, wexp, all0p)
    h1 = _add2(p1)
    p2 = _spmm_ui(meta, wexp, h1)
    amep = _mean3(all0p, h1, p2)
    user_emb = amep[:N_USERS]
    item_emb = amep[N_USERS:N_ALL]

    umeta, uwexp = _pad_edges(uu_edge_index, uu_edge_weight, _E_PAD_UU)
    uep = jnp.pad(user_emb, ((0, N_USERS_PAD - N_USERS), (0, 0)))
    pu = _spmm_uu(umeta, uwexp, uep)
    user_final = _final(user_ego, user_emb, pu[:, :N_USERS], Wm, bm)
    return (user_final, item_emb, user_ego, item_ego)
